# baseline (speedup 1.0000x reference)
import functools

import jax
import jax.numpy as jnp
from jax import lax
from jax.experimental import pallas as pl
from jax.experimental.pallas import tpu as pltpu
from jax.experimental.pallas import tpu_sc as plsc

_MAX_WEIGHT = 20
_HIDDEN = 64
_NC = 2
_NS = 16
_NW = _NC * _NS
_B = 512
_L = 16


def _sc_body(v_hbm, w_hbm, t_hbm, o_hbm, vbuf, widv, tbuf, sem):
    n = v_hbm.shape[0]
    rows_per_w = n // _NW
    chunks = rows_per_w // _B
    wid = lax.axis_index("s") * _NC + lax.axis_index("c")
    w_base = wid * rows_per_w

    pltpu.sync_copy(t_hbm, tbuf)

    @pl.loop(0, chunks)
    def _chunk(ci):
        base = w_base + ci * _B
        pltpu.sync_copy(w_hbm.at[pl.ds(base, _B)], widv)
        pltpu.sync_copy(v_hbm.at[pl.ds(base, _B)], vbuf)

        @pl.loop(0, _B // _L)
        def _add(g):
            wv = widv[pl.ds(g * _L, _L)]
            cl = jnp.maximum(jnp.minimum(wv, _MAX_WEIGHT + 1) - 1, 0)
            for j in range(_L):
                r = g * _L + j
                wr = cl[j]
                for c in range(_HIDDEN // _L):
                    slc = pl.ds(c * _L, _L)
                    vbuf[r, slc] = vbuf[r, slc] + tbuf[wr, slc]

        pltpu.sync_copy(vbuf, o_hbm.at[pl.ds(base, _B)])


def kernel(value_embed, all_weights, weight_embedding):
    n, hidden = value_embed.shape
    mesh = plsc.VectorSubcoreMesh(core_axis_name="c", subcore_axis_name="s")
    sc_kernel = functools.partial(
        pl.kernel,
        out_type=jax.ShapeDtypeStruct((n, hidden), jnp.float32),
        mesh=mesh,
        scratch_types=[
            pltpu.VMEM((_B, hidden), jnp.float32),
            pltpu.VMEM((_B,), jnp.int32),
            pltpu.VMEM((_MAX_WEIGHT + 1, hidden), jnp.float32),
            pltpu.SemaphoreType.DMA,
        ],
    )(_sc_body)
    return sc_kernel(value_embed, all_weights, weight_embedding)


# speedup vs baseline: 1.4571x; 1.4571x over previous
import functools

import jax
import jax.numpy as jnp
from jax import lax
from jax.experimental import pallas as pl
from jax.experimental.pallas import tpu as pltpu
from jax.experimental.pallas import tpu_sc as plsc

_MAX_WEIGHT = 20
_HIDDEN = 64
_NC = 2
_NS = 16
_NW = _NC * _NS
_B = 160
_L = 16
_NBUF = 4


def _sc_body(v_hbm, w_hbm, t_hbm, o_hbm, *scratch):
    vbufs = scratch[0:_NBUF]
    widvs = scratch[_NBUF:2 * _NBUF]
    tbuf = scratch[2 * _NBUF]
    sem_in = scratch[2 * _NBUF + 1]
    sem_out = scratch[2 * _NBUF + 2]

    n = v_hbm.shape[0]
    rows_per_w = n // _NW
    chunks = rows_per_w // _B
    wid = lax.axis_index("s") * _NC + lax.axis_index("c")
    w_base = wid * rows_per_w

    pltpu.sync_copy(t_hbm, tbuf)

    def issue_in(ci, b):
        base = w_base + ci * _B
        pltpu.async_copy(v_hbm.at[pl.ds(base, _B)], vbufs[b], sem_in.at[b])
        pltpu.async_copy(w_hbm.at[pl.ds(base, _B)], widvs[b], sem_in.at[b])

    def wait_in(b):
        pltpu.make_async_copy(v_hbm.at[pl.ds(0, _B)], vbufs[b], sem_in.at[b]).wait()
        pltpu.make_async_copy(w_hbm.at[pl.ds(0, _B)], widvs[b], sem_in.at[b]).wait()

    def issue_out(ci, b):
        base = w_base + ci * _B
        pltpu.async_copy(vbufs[b], o_hbm.at[pl.ds(base, _B)], sem_out.at[b])

    def wait_out(b):
        pltpu.make_async_copy(vbufs[b], o_hbm.at[pl.ds(0, _B)], sem_out.at[b]).wait()

    def compute(b):
        vbuf, widv = vbufs[b], widvs[b]

        @pl.loop(0, _B // _L)
        def _add(g):
            wv = widv[pl.ds(g * _L, _L)]
            cl = jnp.maximum(jnp.minimum(wv, _MAX_WEIGHT + 1) - 1, 0)
            for j in range(_L):
                r = g * _L + j
                wr = cl[j]
                for c in range(_HIDDEN // _L):
                    slc = pl.ds(c * _L, _L)
                    plsc.addupdate(vbuf.at[r, slc], tbuf[wr, slc])

    for b in range(_NBUF - 1):
        issue_in(b, b)

    @pl.loop(0, chunks, step=_NBUF)
    def _ring(c0):
        for b in range(_NBUF):
            ci = c0 + b
            bp = (b - 1) % _NBUF
            wait_in(b)
            compute(b)
            issue_out(ci, b)
            if b == 0:
                @pl.when(ci >= 1)
                def _():
                    wait_out(bp)

                @pl.when(ci + _NBUF - 1 < chunks)
                def _():
                    issue_in(ci + _NBUF - 1, bp)
            else:
                wait_out(bp)

                @pl.when(ci + _NBUF - 1 < chunks)
                def _():
                    issue_in(ci + _NBUF - 1, bp)

    wait_out((chunks - 1) % _NBUF)


def kernel(value_embed, all_weights, weight_embedding):
    n, hidden = value_embed.shape
    mesh = plsc.VectorSubcoreMesh(core_axis_name="c", subcore_axis_name="s")
    scratch = (
        [pltpu.VMEM((_B, hidden), jnp.float32) for _ in range(_NBUF)]
        + [pltpu.VMEM((_B,), jnp.int32) for _ in range(_NBUF)]
        + [
            pltpu.VMEM((_MAX_WEIGHT + 1, hidden), jnp.float32),
            pltpu.SemaphoreType.DMA((_NBUF,)),
            pltpu.SemaphoreType.DMA((_NBUF,)),
        ]
    )
    sc_kernel = functools.partial(
        pl.kernel,
        out_type=jax.ShapeDtypeStruct((n, hidden), jnp.float32),
        mesh=mesh,
        scratch_types=scratch,
    )(_sc_body)
    return sc_kernel(value_embed, all_weights, weight_embedding)
